# baseline (speedup 1.0000x reference)
import jax
import jax.numpy as jnp
from jax.experimental import pallas as pl

_N = 16384
_D = 64
_K = 4096
_RT = 128
_W = 32


def _norms_body(x_ref, y_ref, ox_ref, oy_ref):
    for ref, out in ((x_ref, ox_ref), (y_ref, oy_ref)):
        t = ref[...]
        sq = (t * t).reshape(t.shape[0], 8, 8)
        P = sq[:, 0, :]
        for g in range(1, 8):
            P = P + sq[:, g, :]
        A1 = P[:, 0:4] + P[:, 4:8]
        A2 = A1[:, 0:2] + A1[:, 2:4]
        out[...] = A2[:, 0] + A2[:, 1]


def _norms(feat_select, feat_target):
    rt = 1024
    return pl.pallas_call(
        _norms_body,
        grid=(_N // rt,),
        in_specs=[pl.BlockSpec((rt, _D), lambda i: (i, 0)),
                  pl.BlockSpec((rt, _D), lambda i: (i, 0))],
        out_specs=[pl.BlockSpec((rt,), lambda i: (i,)),
                   pl.BlockSpec((rt,), lambda i: (i,))],
        out_shape=[jax.ShapeDtypeStruct((_N,), jnp.float32),
                   jax.ShapeDtypeStruct((_N,), jnp.float32)],
    )(feat_select, feat_target)


def _metric_body(x_ref, x2_ref, y_ref, y2_ref, o_ref):
    xt = x_ref[...]
    x2 = x2_ref[...][None, :]
    yt = y_ref[...]
    y2 = y2_ref[...][:, None]
    xy = jnp.dot(yt, xt.T, preferred_element_type=jnp.float32)
    d2 = (x2 + y2) - 2.0 * xy
    dist = jnp.sqrt(jnp.maximum(d2, 1e-12))
    D4 = dist.reshape(_W, (_N // _W) // 8, 8, _RT)
    P = jnp.sum(D4, axis=1)
    A1 = P[:, 0:4, :] + P[:, 4:8, :]
    A2 = A1[:, 0:2, :] + A1[:, 2:4, :]
    A3 = A2[:, 0, :] + A2[:, 1, :]
    m = A3[0]
    for w in range(1, _W):
        m = m + A3[w]
    o_ref[...] = m


def _metric(feat_select, feat_target, x2, y2):
    return pl.pallas_call(
        _metric_body,
        grid=(_N // _RT,),
        in_specs=[pl.BlockSpec((_RT, _D), lambda i: (i, 0)),
                  pl.BlockSpec((_RT,), lambda i: (i,)),
                  pl.BlockSpec((_N, _D), lambda i: (0, 0)),
                  pl.BlockSpec((_N,), lambda i: (0,))],
        out_specs=pl.BlockSpec((_RT,), lambda i: (i,)),
        out_shape=jax.ShapeDtypeStruct((_N,), jnp.float32),
    )(feat_select, x2, feat_target, y2)


def kernel(feat_select, feat_target, k):
    x2, y2 = _norms(feat_select, feat_target)
    m = _metric(feat_select, feat_target, x2, y2)
    _, idx = jax.lax.top_k(m, _K)
    selected = jnp.take(feat_select, idx, axis=0)
    return selected, idx
